# baseline (speedup 1.0000x reference)
import functools
import math

import jax
import jax.numpy as jnp
from jax import lax
from jax.experimental import pallas as pl
from jax.experimental.pallas import tpu as pltpu
from jax.experimental.pallas import tpu_sc as plsc

N = 10000
NP = 10240
E = 320000
NW = 32
EPT = 2 * E // NW
C = 128
NCH = 160
EPTP = NCH * C
G = C // 16
TRASH = NP - 1
NPW = NP // 16
PB = 2048


def _proj(h, wk, bk, wq, bq, wv, bv, ak, mv, ps):
    k = (h @ wk + bk) @ ak * ps
    q = h @ wq + bq
    v = (h @ wv + bv) @ mv
    return k, q, v


def _prep_body(x_ref, wk_ref, bk_ref, wq_ref, bq_ref, wv_ref, bv_ref,
               ak_ref, mv_ref, ps_ref, k_ref, q_ref, v_ref):
    k, q, vv = _proj(x_ref[...], wk_ref[...], bk_ref[...], wq_ref[...],
                     bq_ref[...], wv_ref[...], bv_ref[...], ak_ref[...],
                     mv_ref[...], ps_ref[0, 0])
    k_ref[...] = k
    q_ref[...] = q
    v_ref[...] = vv


def _mid_body(g_ref, wa_ref, ba_ref, wk_ref, bk_ref, wq_ref, bq_ref,
              wv_ref, bv_ref, ak_ref, mv_ref, ps_ref, k_ref, q_ref, v_ref):
    p = g_ref[...]
    d = wa_ref.shape[0]
    h = jax.nn.gelu(p[:, :d] / (p[:, d:d + 1] + 1e-16)) @ wa_ref[...] + ba_ref[...]
    k, q, vv = _proj(h, wk_ref[...], bk_ref[...], wq_ref[...], bq_ref[...],
                     wv_ref[...], bv_ref[...], ak_ref[...], mv_ref[...],
                     ps_ref[0, 0])
    k_ref[...] = k
    q_ref[...] = q
    v_ref[...] = vv


def _fin_body(g_ref, wa_ref, ba_ref, o_ref):
    p = g_ref[...]
    d = wa_ref.shape[0]
    o_ref[...] = jax.nn.gelu(p[:, :d] / (p[:, d:d + 1] + 1e-16)) @ wa_ref[...] + ba_ref[...]


def _rep(shape):
    return pl.BlockSpec(shape, lambda i: (0, 0))


def _proj_specs(din, dout):
    return [
        _rep((din, dout)), _rep((1, dout)),
        _rep((din, dout)), _rep((1, dout)),
        _rep((din, dout)), _rep((1, dout)),
        _rep((dout, dout)), _rep((dout, dout)),
        _rep((1, 1)),
    ]


def _proj_args(p, nt, rel, dout):
    wk, bk = p["k_" + nt]
    wq, bq = p["q_" + nt]
    wv, bv = p["v_" + nt]
    ps = (p["prel_" + rel] * (1.0 / math.sqrt(dout))).reshape(1, 1)
    return (wk, bk.reshape(1, -1), wq, bq.reshape(1, -1), wv, bv.reshape(1, -1),
            p["arel_" + rel], p["mrel_" + rel], ps)


def _prep(x_pad, p, nt, rel, din, dout):
    return pl.pallas_call(
        _prep_body,
        grid=(NP // PB,),
        in_specs=[pl.BlockSpec((PB, din), lambda i: (i, 0))] + _proj_specs(din, dout),
        out_specs=[pl.BlockSpec((PB, dout), lambda i: (i, 0))] * 3,
        out_shape=[jax.ShapeDtypeStruct((NP, dout), jnp.float32)] * 3,
    )(x_pad, *_proj_args(p, nt, rel, dout))


def _mid(agg, p1, p2, nt, rel, dmid, dout):
    wa, ba = p1["a_" + nt]
    return pl.pallas_call(
        _mid_body,
        grid=(NP // PB,),
        in_specs=[pl.BlockSpec((PB, dmid + 8), lambda i: (i, 0)),
                  _rep((dmid, dmid)), _rep((1, dmid))] + _proj_specs(dmid, dout),
        out_specs=[pl.BlockSpec((PB, dout), lambda i: (i, 0))] * 3,
        out_shape=[jax.ShapeDtypeStruct((NP, dout), jnp.float32)] * 3,
    )(agg, wa, ba.reshape(1, -1), *_proj_args(p2, nt, rel, dout))


def _fin(agg, p, nt, dout):
    wa, ba = p["a_" + nt]
    return pl.pallas_call(
        _fin_body,
        grid=(NP // PB,),
        in_specs=[pl.BlockSpec((PB, dout + 8), lambda i: (i, 0)),
                  _rep((dout, dout)), _rep((1, dout))],
        out_specs=pl.BlockSpec((PB, dout), lambda i: (i, 0)),
        out_shape=jax.ShapeDtypeStruct((NP, dout), jnp.float32),
    )(agg, wa, ba.reshape(1, -1))


def _edge_sweep(dout, nstage):
    row = dout + 8
    S = NCH // nstage
    mesh = plsc.VectorSubcoreMesh(core_axis_name="c", subcore_axis_name="s")

    def body(ei_ref, kt_ref, qt_ref, z_ref, out_ref,
             idxall, kb, qb, vo, sh, sem_k, sem_q):
        cid = lax.axis_index("c")
        sid = lax.axis_index("s")
        gid = cid * 16 + sid
        pltpu.sync_copy(z_ref, sh.at[pl.ds(sid * NPW, NPW)])
        plsc.subcore_barrier()
        lanes = lax.iota(jnp.int32, 16)

        for jc in range(dout + 1, row):
            for g in range(G):
                plsc.store_scatter(vo,
                                   [lanes + g * 16,
                                    jnp.full((16,), jc, jnp.int32)],
                                   jnp.zeros((16,), jnp.float32))

        for st in range(nstage):
            pltpu.sync_copy(ei_ref.at[gid, pl.ds(st * S, S)], idxall)

            def chunk(j, carry):
                ck = pltpu.async_copy(kt_ref.at[idxall.at[j, 0]], kb, sem_k)
                cq = pltpu.async_copy(qt_ref.at[idxall.at[j, 1]], qb, sem_q)
                ck.wait()
                cq.wait()
                def group(g, carry):
                    rows = lanes + g * 16
                    acc = jnp.zeros((16,), jnp.float32)
                    for j2 in range(dout):
                        cols = jnp.full((16,), j2, jnp.int32)
                        acc = acc + (plsc.load_gather(kb, [rows, cols]) *
                                     plsc.load_gather(qb, [rows, cols]))
                    w = jnp.exp(acc)
                    for j2 in range(dout):
                        cols = jnp.full((16,), j2, jnp.int32)
                        plsc.store_scatter(
                            vo, [rows, cols],
                            plsc.load_gather(kb, [rows, cols + dout]) * w)
                    plsc.store_scatter(
                        vo, [rows, jnp.full((16,), dout, jnp.int32)], w)
                    return carry

                lax.fori_loop(0, G, group, 0)
                pltpu.sync_copy(vo, sh.at[idxall.at[j, 2]], add=True)
                return carry

            lax.fori_loop(0, S, chunk, 0)
        plsc.subcore_barrier()
        pltpu.sync_copy(sh.at[pl.ds(sid * NPW, NPW)],
                        out_ref.at[cid, pl.ds(sid * NPW, NPW)])

    return pl.kernel(
        body,
        out_type=jax.ShapeDtypeStruct((2, NP, row), jnp.float32),
        mesh=mesh,
        compiler_params=pltpu.CompilerParams(use_tc_tiling_on_sc=False,
                                             needs_layout_passes=False),
        scratch_types=[
            pltpu.VMEM((S, 3, C), jnp.int32),
            pltpu.VMEM((C, 2 * dout), jnp.float32),
            pltpu.VMEM((C, dout), jnp.float32),
            pltpu.VMEM((C, row), jnp.float32),
            pltpu.VMEM_SHARED((NP, row), jnp.float32),
            pltpu.SemaphoreType.DMA,
            pltpu.SemaphoreType.DMA,
        ])


def kernel(x_paper, x_author, edge_index_writes, edge_index_rev_writes,
           params1, params2):
    xp = jnp.pad(x_paper, ((0, NP - N), (0, 0)))
    xa = jnp.pad(x_author, ((0, NP - N), (0, 0)))
    eiw = edge_index_writes
    eir = edge_index_rev_writes
    ei3 = jnp.stack([
        jnp.concatenate([eiw[0], eir[0] + NP]),
        jnp.concatenate([eiw[1], eir[1] + NP]),
        jnp.concatenate([eiw[1], eir[1]]),
    ]).reshape(3, NW, EPT)
    padv = jnp.stack([jnp.zeros((EPTP - EPT,), jnp.int32),
                      jnp.zeros((EPTP - EPT,), jnp.int32),
                      jnp.full((EPTP - EPT,), TRASH, jnp.int32)])
    ei3 = jnp.concatenate(
        [ei3, jnp.broadcast_to(padv[:, None, :], (3, NW, EPTP - EPT))],
        axis=2).reshape(3, NW, NCH, C).transpose(1, 2, 0, 3)
    z40 = jnp.zeros((NPW, 40), jnp.float32)
    z72 = jnp.zeros((NPW, 72), jnp.float32)

    ka, qa, va = _prep(xa, params1, "author", "writes", 128, 32)
    kp, qp, vp = _prep(xp, params1, "paper", "rev_writes", 128, 32)
    kt = jnp.concatenate([jnp.concatenate([ka, va], axis=1),
                          jnp.concatenate([kp, vp], axis=1)])
    qt = jnp.concatenate([qp, qa])
    agg1 = _edge_sweep(32, 1)(ei3, kt, qt, z40)

    k2p, q2p, v2p = _mid(agg1[0], params1, params2, "paper", "rev_writes", 32, 64)
    k2a, q2a, v2a = _mid(agg1[1], params1, params2, "author", "writes", 32, 64)
    kt2 = jnp.concatenate([jnp.concatenate([k2a, v2a], axis=1),
                           jnp.concatenate([k2p, v2p], axis=1)])
    qt2 = jnp.concatenate([q2p, q2a])
    agg2 = _edge_sweep(64, 2)(ei3, kt2, qt2, z72)

    out_p = _fin(agg2[0], params2, "paper", 64)[:N]
    out_a = _fin(agg2[1], params2, "author", 64)[:N]
    return (out_p, out_a)


# speedup vs baseline: 1.0755x; 1.0755x over previous
import functools
import math

import jax
import jax.numpy as jnp
from jax import lax
from jax.experimental import pallas as pl
from jax.experimental.pallas import tpu as pltpu
from jax.experimental.pallas import tpu_sc as plsc

N = 10000
NP = 10240
E = 320000
NW = 32
EPT = 2 * E // NW
C = 128
NCH = 160
EPTP = NCH * C
G = C // 16
TRASH = NP - 1
NPW = NP // 16
PB = 2048


def _proj(h, wk, bk, wq, bq, wv, bv, ak, mv, ps):
    k = (h @ wk + bk) @ ak * ps
    q = h @ wq + bq
    v = (h @ wv + bv) @ mv
    return k, q, v


def _prep_body(x_ref, wk_ref, bk_ref, wq_ref, bq_ref, wv_ref, bv_ref,
               ak_ref, mv_ref, ps_ref, k_ref, q_ref, v_ref):
    k, q, vv = _proj(x_ref[...], wk_ref[...], bk_ref[...], wq_ref[...],
                     bq_ref[...], wv_ref[...], bv_ref[...], ak_ref[...],
                     mv_ref[...], ps_ref[0, 0])
    k_ref[...] = k
    q_ref[...] = q
    v_ref[...] = vv


def _mid_body(g_ref, wa_ref, ba_ref, wk_ref, bk_ref, wq_ref, bq_ref,
              wv_ref, bv_ref, ak_ref, mv_ref, ps_ref, k_ref, q_ref, v_ref):
    p = g_ref[...]
    d = wa_ref.shape[0]
    h = jax.nn.gelu(p[:, :d] / (p[:, d:d + 1] + 1e-16)) @ wa_ref[...] + ba_ref[...]
    k, q, vv = _proj(h, wk_ref[...], bk_ref[...], wq_ref[...], bq_ref[...],
                     wv_ref[...], bv_ref[...], ak_ref[...], mv_ref[...],
                     ps_ref[0, 0])
    k_ref[...] = k
    q_ref[...] = q
    v_ref[...] = vv


def _fin_body(g_ref, wa_ref, ba_ref, o_ref):
    p = g_ref[...]
    d = wa_ref.shape[0]
    o_ref[...] = jax.nn.gelu(p[:, :d] / (p[:, d:d + 1] + 1e-16)) @ wa_ref[...] + ba_ref[...]


def _rep(shape):
    return pl.BlockSpec(shape, lambda i: (0, 0))


def _proj_specs(din, dout):
    return [
        _rep((din, dout)), _rep((1, dout)),
        _rep((din, dout)), _rep((1, dout)),
        _rep((din, dout)), _rep((1, dout)),
        _rep((dout, dout)), _rep((dout, dout)),
        _rep((1, 1)),
    ]


def _proj_args(p, nt, rel, dout):
    wk, bk = p["k_" + nt]
    wq, bq = p["q_" + nt]
    wv, bv = p["v_" + nt]
    ps = (p["prel_" + rel] * (1.0 / math.sqrt(dout))).reshape(1, 1)
    return (wk, bk.reshape(1, -1), wq, bq.reshape(1, -1), wv, bv.reshape(1, -1),
            p["arel_" + rel], p["mrel_" + rel], ps)


def _prep(x_pad, p, nt, rel, din, dout):
    return pl.pallas_call(
        _prep_body,
        grid=(NP // PB,),
        in_specs=[pl.BlockSpec((PB, din), lambda i: (i, 0))] + _proj_specs(din, dout),
        out_specs=[pl.BlockSpec((PB, dout), lambda i: (i, 0))] * 3,
        out_shape=[jax.ShapeDtypeStruct((NP, dout), jnp.float32)] * 3,
    )(x_pad, *_proj_args(p, nt, rel, dout))


def _mid(agg, p1, p2, nt, rel, dmid, dout):
    wa, ba = p1["a_" + nt]
    return pl.pallas_call(
        _mid_body,
        grid=(NP // PB,),
        in_specs=[pl.BlockSpec((PB, dmid + 8), lambda i: (i, 0)),
                  _rep((dmid, dmid)), _rep((1, dmid))] + _proj_specs(dmid, dout),
        out_specs=[pl.BlockSpec((PB, dout), lambda i: (i, 0))] * 3,
        out_shape=[jax.ShapeDtypeStruct((NP, dout), jnp.float32)] * 3,
    )(agg, wa, ba.reshape(1, -1), *_proj_args(p2, nt, rel, dout))


def _fin(agg, p, nt, dout):
    wa, ba = p["a_" + nt]
    return pl.pallas_call(
        _fin_body,
        grid=(NP // PB,),
        in_specs=[pl.BlockSpec((PB, dout + 8), lambda i: (i, 0)),
                  _rep((dout, dout)), _rep((1, dout))],
        out_specs=pl.BlockSpec((PB, dout), lambda i: (i, 0)),
        out_shape=jax.ShapeDtypeStruct((NP, dout), jnp.float32),
    )(agg, wa, ba.reshape(1, -1))


def _edge_sweep(dout, nstage):
    row = dout + 8
    S = NCH // nstage
    mesh = plsc.VectorSubcoreMesh(core_axis_name="c", subcore_axis_name="s")

    def body(ei_ref, kt_ref, qt_ref, vt_ref, z_ref, out_ref,
             idxall, kb, qb, vi, vo, sh, sem_k, sem_q, sem_v):
        cid = lax.axis_index("c")
        sid = lax.axis_index("s")
        gid = cid * 16 + sid
        pltpu.sync_copy(z_ref, sh.at[pl.ds(sid * NPW, NPW)])
        plsc.subcore_barrier()
        lanes = lax.iota(jnp.int32, 16)

        for jc in range(dout + 1, row):
            for g in range(G):
                plsc.store_scatter(vo,
                                   [lanes + g * 16,
                                    jnp.full((16,), jc, jnp.int32)],
                                   jnp.zeros((16,), jnp.float32))

        for st in range(nstage):
            pltpu.sync_copy(ei_ref.at[gid, pl.ds(st * S, S)], idxall)

            def chunk(j, carry):
                ck = pltpu.async_copy(kt_ref.at[idxall.at[j, 0]], kb, sem_k)
                cq = pltpu.async_copy(qt_ref.at[idxall.at[j, 1]], qb, sem_q)
                cv = pltpu.async_copy(vt_ref.at[idxall.at[j, 0]], vi, sem_v)
                ck.wait()
                cq.wait()
                cv.wait()
                def gpair(h, carry):
                    for gg in range(2):
                        rows = lanes + h * 32 + gg * 16
                        accs = [jnp.zeros((16,), jnp.float32)
                                for _ in range(8)]
                        for j2 in range(dout):
                            cols = jnp.full((16,), j2, jnp.int32)
                            accs[j2 % 8] = accs[j2 % 8] + (
                                plsc.load_gather(kb, [rows, cols]) *
                                plsc.load_gather(qb, [rows, cols]))
                        a0 = (accs[0] + accs[1]) + (accs[2] + accs[3])
                        a1 = (accs[4] + accs[5]) + (accs[6] + accs[7])
                        w = jnp.exp(a0 + a1)
                        for j2 in range(dout):
                            cols = jnp.full((16,), j2, jnp.int32)
                            plsc.store_scatter(
                                vo, [rows, cols],
                                plsc.load_gather(vi, [rows, cols]) * w)
                        plsc.store_scatter(
                            vo, [rows, jnp.full((16,), dout, jnp.int32)], w)
                    return carry

                lax.fori_loop(0, G // 2, gpair, 0)
                pltpu.sync_copy(vo, sh.at[idxall.at[j, 2]], add=True)
                return carry

            lax.fori_loop(0, S, chunk, 0)
        plsc.subcore_barrier()
        pltpu.sync_copy(sh.at[pl.ds(sid * NPW, NPW)],
                        out_ref.at[cid, pl.ds(sid * NPW, NPW)])

    return pl.kernel(
        body,
        out_type=jax.ShapeDtypeStruct((2, NP, row), jnp.float32),
        mesh=mesh,
        compiler_params=pltpu.CompilerParams(use_tc_tiling_on_sc=False,
                                             needs_layout_passes=False),
        scratch_types=[
            pltpu.VMEM((S, 3, C), jnp.int32),
            pltpu.VMEM((C, dout), jnp.float32),
            pltpu.VMEM((C, dout), jnp.float32),
            pltpu.VMEM((C, dout), jnp.float32),
            pltpu.VMEM((C, row), jnp.float32),
            pltpu.VMEM_SHARED((NP, row), jnp.float32),
            pltpu.SemaphoreType.DMA,
            pltpu.SemaphoreType.DMA,
            pltpu.SemaphoreType.DMA,
        ])


def kernel(x_paper, x_author, edge_index_writes, edge_index_rev_writes,
           params1, params2):
    xp = jnp.pad(x_paper, ((0, NP - N), (0, 0)))
    xa = jnp.pad(x_author, ((0, NP - N), (0, 0)))
    eiw = edge_index_writes
    eir = edge_index_rev_writes
    ei3 = jnp.stack([
        jnp.concatenate([eiw[0], eir[0] + NP]),
        jnp.concatenate([eiw[1], eir[1] + NP]),
        jnp.concatenate([eiw[1], eir[1]]),
    ]).reshape(3, NW, EPT)
    padv = jnp.stack([jnp.zeros((EPTP - EPT,), jnp.int32),
                      jnp.zeros((EPTP - EPT,), jnp.int32),
                      jnp.full((EPTP - EPT,), TRASH, jnp.int32)])
    ei3 = jnp.concatenate(
        [ei3, jnp.broadcast_to(padv[:, None, :], (3, NW, EPTP - EPT))],
        axis=2).reshape(3, NW, NCH, C).transpose(1, 2, 0, 3)
    z40 = jnp.zeros((NPW, 40), jnp.float32)
    z72 = jnp.zeros((NPW, 72), jnp.float32)

    ka, qa, va = _prep(xa, params1, "author", "writes", 128, 32)
    kp, qp, vp = _prep(xp, params1, "paper", "rev_writes", 128, 32)
    kt = jnp.concatenate([ka, kp])
    qt = jnp.concatenate([qp, qa])
    vt = jnp.concatenate([va, vp])
    agg1 = _edge_sweep(32, 2)(ei3, kt, qt, vt, z40)

    k2p, q2p, v2p = _mid(agg1[0], params1, params2, "paper", "rev_writes", 32, 64)
    k2a, q2a, v2a = _mid(agg1[1], params1, params2, "author", "writes", 32, 64)
    kt2 = jnp.concatenate([k2a, k2p])
    qt2 = jnp.concatenate([q2p, q2a])
    vt2 = jnp.concatenate([v2a, v2p])
    agg2 = _edge_sweep(64, 4)(ei3, kt2, qt2, vt2, z72)

    out_p = _fin(agg2[0], params2, "paper", 64)[:N]
    out_a = _fin(agg2[1], params2, "author", 64)[:N]
    return (out_p, out_a)


# speedup vs baseline: 1.3360x; 1.2423x over previous
import math

import jax
import jax.numpy as jnp
from jax import lax
from jax.experimental import pallas as pl
from jax.experimental.pallas import tpu as pltpu
from jax.experimental.pallas import tpu_sc as plsc

N = 10000
NP = 10240
E = 320000
NW = 32
EPT = 2 * E // NW
C = 128
NCH = (EPT + C - 1) // C
EPTP = NCH * C
G = C // 16
TRASH = NP - 1
NPW = NP // 16
PB = 2048


def _proj(h, wk, bk, wq, bq, wv, bv, ak, mv, ps):
    k = (h @ wk + bk) @ ak * ps
    q = h @ wq + bq
    v = (h @ wv + bv) @ mv
    blk = v.shape[0]
    vv = jnp.concatenate(
        [v, jnp.ones((blk, 1), jnp.float32), jnp.zeros((blk, 7), jnp.float32)],
        axis=1)
    return k, q, vv


def _prep_body(x_ref, wk_ref, bk_ref, wq_ref, bq_ref, wv_ref, bv_ref,
               ak_ref, mv_ref, ps_ref, k_ref, q_ref, v_ref):
    k, q, vv = _proj(x_ref[...], wk_ref[...], bk_ref[...], wq_ref[...],
                     bq_ref[...], wv_ref[...], bv_ref[...], ak_ref[...],
                     mv_ref[...], ps_ref[0, 0])
    k_ref[...] = k
    q_ref[...] = q
    v_ref[...] = vv


def _mid_body(g_ref, wa_ref, ba_ref, wk_ref, bk_ref, wq_ref, bq_ref,
              wv_ref, bv_ref, ak_ref, mv_ref, ps_ref, k_ref, q_ref, v_ref):
    p = g_ref[...]
    d = wa_ref.shape[0]
    h = jax.nn.gelu(p[:, :d] / (p[:, d:d + 1] + 1e-16)) @ wa_ref[...] + ba_ref[...]
    k, q, vv = _proj(h, wk_ref[...], bk_ref[...], wq_ref[...], bq_ref[...],
                     wv_ref[...], bv_ref[...], ak_ref[...], mv_ref[...],
                     ps_ref[0, 0])
    k_ref[...] = k
    q_ref[...] = q
    v_ref[...] = vv


def _fin_body(g_ref, wa_ref, ba_ref, o_ref):
    p = g_ref[...]
    d = wa_ref.shape[0]
    o_ref[...] = jax.nn.gelu(p[:, :d] / (p[:, d:d + 1] + 1e-16)) @ wa_ref[...] + ba_ref[...]


def _rep(shape):
    return pl.BlockSpec(shape, lambda i: (0, 0))


def _proj_specs(din, dout):
    return [
        _rep((din, dout)), _rep((1, dout)),
        _rep((din, dout)), _rep((1, dout)),
        _rep((din, dout)), _rep((1, dout)),
        _rep((dout, dout)), _rep((dout, dout)),
        _rep((1, 1)),
    ]


def _proj_args(p, nt, rel, dout):
    wk, bk = p["k_" + nt]
    wq, bq = p["q_" + nt]
    wv, bv = p["v_" + nt]
    ps = (p["prel_" + rel] * (1.0 / math.sqrt(dout))).reshape(1, 1)
    return (wk, bk.reshape(1, -1), wq, bq.reshape(1, -1), wv, bv.reshape(1, -1),
            p["arel_" + rel], p["mrel_" + rel], ps)


def _prep(x_pad, p, nt, rel, din, dout):
    row = dout + 8
    return pl.pallas_call(
        _prep_body,
        grid=(NP // PB,),
        in_specs=[pl.BlockSpec((PB, din), lambda i: (i, 0))] + _proj_specs(din, dout),
        out_specs=[pl.BlockSpec((PB, dout), lambda i: (i, 0)),
                   pl.BlockSpec((PB, dout), lambda i: (i, 0)),
                   pl.BlockSpec((PB, row), lambda i: (i, 0))],
        out_shape=[jax.ShapeDtypeStruct((NP, dout), jnp.float32),
                   jax.ShapeDtypeStruct((NP, dout), jnp.float32),
                   jax.ShapeDtypeStruct((NP, row), jnp.float32)],
    )(x_pad, *_proj_args(p, nt, rel, dout))


def _mid(agg, p1, p2, nt, rel, dmid, dout):
    row = dout + 8
    wa, ba = p1["a_" + nt]
    return pl.pallas_call(
        _mid_body,
        grid=(NP // PB,),
        in_specs=[pl.BlockSpec((PB, dmid + 8), lambda i: (i, 0)),
                  _rep((dmid, dmid)), _rep((1, dmid))] + _proj_specs(dmid, dout),
        out_specs=[pl.BlockSpec((PB, dout), lambda i: (i, 0)),
                   pl.BlockSpec((PB, dout), lambda i: (i, 0)),
                   pl.BlockSpec((PB, row), lambda i: (i, 0))],
        out_shape=[jax.ShapeDtypeStruct((NP, dout), jnp.float32),
                   jax.ShapeDtypeStruct((NP, dout), jnp.float32),
                   jax.ShapeDtypeStruct((NP, row), jnp.float32)],
    )(agg, wa, ba.reshape(1, -1), *_proj_args(p2, nt, rel, dout))


def _fin(agg, p, nt, dout):
    wa, ba = p["a_" + nt]
    return pl.pallas_call(
        _fin_body,
        grid=(NP // PB,),
        in_specs=[pl.BlockSpec((PB, dout + 8), lambda i: (i, 0)),
                  _rep((dout, dout)), _rep((1, dout))],
        out_specs=pl.BlockSpec((PB, dout), lambda i: (i, 0)),
        out_shape=jax.ShapeDtypeStruct((NP, dout), jnp.float32),
    )(agg, wa, ba.reshape(1, -1))


def _edge_sweep(dout):
    row = dout + 8
    mesh = plsc.VectorSubcoreMesh(core_axis_name="c", subcore_axis_name="s")

    def body(ei_ref, kt_ref, qt_ref, vt_ref, z_ref, out_ref,
             idx_ref, kb, qb, vb, sh, sem_k, sem_q, sem_v):
        cid = lax.axis_index("c")
        sid = lax.axis_index("s")
        gid = cid * 16 + sid
        pltpu.sync_copy(z_ref, sh.at[pl.ds(sid * NPW, NPW)])
        plsc.subcore_barrier()
        lanes = lax.iota(jnp.int32, 16)

        def chunk(i, carry):
            pltpu.sync_copy(ei_ref.at[gid, :, pl.ds(i * C, C)], idx_ref)
            ck = pltpu.async_copy(kt_ref.at[idx_ref.at[0]], kb, sem_k)
            cq = pltpu.async_copy(qt_ref.at[idx_ref.at[1]], qb, sem_q)
            cv = pltpu.async_copy(vt_ref.at[idx_ref.at[0]], vb, sem_v)
            ck.wait()
            cq.wait()
            cv.wait()
            for g in range(G):
                rows = lanes + (g * 16)
                acc = jnp.zeros((16,), jnp.float32)
                for j in range(dout):
                    cols = jnp.full((16,), j, jnp.int32)
                    acc = acc + (plsc.load_gather(kb, [rows, cols]) *
                                 plsc.load_gather(qb, [rows, cols]))
                w = jnp.exp(acc)
                for j in range(dout + 1):
                    cols = jnp.full((16,), j, jnp.int32)
                    plsc.store_scatter(
                        vb, [rows, cols],
                        plsc.load_gather(vb, [rows, cols]) * w)
            pltpu.sync_copy(vb, sh.at[idx_ref.at[2]], add=True)
            return carry

        lax.fori_loop(0, NCH, chunk, 0)
        plsc.subcore_barrier()
        pltpu.sync_copy(sh.at[pl.ds(sid * NPW, NPW)],
                        out_ref.at[cid, pl.ds(sid * NPW, NPW)])

    return pl.kernel(
        body,
        out_type=jax.ShapeDtypeStruct((2, NP, row), jnp.float32),
        mesh=mesh,
        compiler_params=pltpu.CompilerParams(use_tc_tiling_on_sc=False,
                                             needs_layout_passes=False),
        scratch_types=[
            pltpu.VMEM((3, C), jnp.int32),
            pltpu.VMEM((C, dout), jnp.float32),
            pltpu.VMEM((C, dout), jnp.float32),
            pltpu.VMEM((C, row), jnp.float32),
            pltpu.VMEM_SHARED((NP, row), jnp.float32),
            pltpu.SemaphoreType.DMA,
            pltpu.SemaphoreType.DMA,
            pltpu.SemaphoreType.DMA,
        ])


def kernel(x_paper, x_author, edge_index_writes, edge_index_rev_writes,
           params1, params2):
    xp = jnp.pad(x_paper, ((0, NP - N), (0, 0)))
    xa = jnp.pad(x_author, ((0, NP - N), (0, 0)))
    eiw = edge_index_writes
    eir = edge_index_rev_writes
    ei3 = jnp.stack([
        jnp.concatenate([eiw[0], eir[0] + NP]),
        jnp.concatenate([eiw[1], eir[1] + NP]),
        jnp.concatenate([eiw[1], eir[1]]),
    ]).reshape(3, NW, EPT)
    padv = jnp.stack([jnp.zeros((EPTP - EPT,), jnp.int32),
                      jnp.zeros((EPTP - EPT,), jnp.int32),
                      jnp.full((EPTP - EPT,), TRASH, jnp.int32)])
    ei3 = jnp.concatenate(
        [ei3, jnp.broadcast_to(padv[:, None, :], (3, NW, EPTP - EPT))],
        axis=2).transpose(1, 0, 2)
    z40 = jnp.zeros((NPW, 40), jnp.float32)
    z72 = jnp.zeros((NPW, 72), jnp.float32)

    ka, qa, va = _prep(xa, params1, "author", "writes", 128, 32)
    kp, qp, vp = _prep(xp, params1, "paper", "rev_writes", 128, 32)
    kt = jnp.concatenate([ka, kp])
    qt = jnp.concatenate([qp, qa])
    vt = jnp.concatenate([va, vp])
    agg1 = _edge_sweep(32)(ei3, kt, qt, vt, z40)

    k2p, q2p, v2p = _mid(agg1[0], params1, params2, "paper", "rev_writes", 32, 64)
    k2a, q2a, v2a = _mid(agg1[1], params1, params2, "author", "writes", 32, 64)
    kt2 = jnp.concatenate([k2a, k2p])
    qt2 = jnp.concatenate([q2p, q2a])
    vt2 = jnp.concatenate([v2a, v2p])
    agg2 = _edge_sweep(64)(ei3, kt2, qt2, vt2, z72)

    out_p = _fin(agg2[0], params2, "paper", 64)[:N]
    out_a = _fin(agg2[1], params2, "author", 64)[:N]
    return (out_p, out_a)
